# baseline (speedup 1.0000x reference)
import functools

import jax
import jax.numpy as jnp
from jax.experimental import pallas as pl
from jax.experimental.pallas import tpu as pltpu

B, T, C = 256, 32, 128
NH, HD = 4, 32
NE, TOPK, DFF = 8, 2, 512

BB = 8
P = BB * T


def _fused_kernel(x_ref, eps_ref, wq_ref, wk_ref, wv_ref, wp_ref, bp_ref,
                  ln1g_ref, ln1b_ref, ln2g_ref, ln2b_ref,
                  wer_ref, ber_ref, wn_ref, bn_ref, ws_ref, bs_ref,
                  we1_ref, be1_ref, we2_ref, be2_ref, out_ref):
    xb = x_ref[...]
    x2 = xb.reshape(P, C)

    m = jnp.mean(x2, axis=-1, keepdims=True)
    v = jnp.mean((x2 - m) ** 2, axis=-1, keepdims=True)
    xn = (x2 - m) * jax.lax.rsqrt(v + 1e-5) * ln1g_ref[...] + ln1b_ref[...]

    scale = C ** -0.5
    q = jnp.dot(xn, wq_ref[...], preferred_element_type=jnp.float32)
    k = jnp.dot(xn, wk_ref[...], preferred_element_type=jnp.float32)
    vv = jnp.dot(xn, wv_ref[...], preferred_element_type=jnp.float32)

    row = jax.lax.broadcasted_iota(jnp.int32, (BB, T, T), 1)
    col = jax.lax.broadcasted_iota(jnp.int32, (BB, T, T), 2)
    causal = row >= col

    att_cols = []
    for h in range(NH):
        qh = q[:, h * HD:(h + 1) * HD].reshape(BB, T, HD)
        kh = k[:, h * HD:(h + 1) * HD].reshape(BB, T, HD)
        vh = vv[:, h * HD:(h + 1) * HD].reshape(BB, T, HD)
        s = jax.lax.dot_general(
            qh, kh, (((2,), (2,)), ((0,), (0,))),
            preferred_element_type=jnp.float32) * scale
        s = jnp.where(causal, s, -1e30)
        s = s - jnp.max(s, axis=-1, keepdims=True)
        e = jnp.exp(s)
        w = e / jnp.sum(e, axis=-1, keepdims=True)
        ah = jax.lax.dot_general(
            w, vh, (((2,), (1,)), ((0,), (0,))),
            preferred_element_type=jnp.float32)
        att_cols.append(ah.reshape(P, HD))
    att = jnp.concatenate(att_cols, axis=-1)

    x1 = x2 + jnp.dot(att, wp_ref[...], preferred_element_type=jnp.float32) \
        + bp_ref[...]

    m2 = jnp.mean(x1, axis=-1, keepdims=True)
    v2 = jnp.mean((x1 - m2) ** 2, axis=-1, keepdims=True)
    xn2 = (x1 - m2) * jax.lax.rsqrt(v2 + 1e-5) * ln2g_ref[...] + ln2b_ref[...]

    logits = jnp.dot(xn2, wer_ref[...], preferred_element_type=jnp.float32) \
        + ber_ref[...]
    nlog = jnp.dot(xn2, wn_ref[...], preferred_element_type=jnp.float32) \
        + bn_ref[...]
    eps = eps_ref[...].reshape(P, NE)
    noisy = logits + eps * jax.nn.softplus(nlog)

    idx = jax.lax.broadcasted_iota(jnp.int32, (P, NE), 1)
    v1 = jnp.max(noisy, axis=-1, keepdims=True)
    i1 = jnp.min(jnp.where(noisy == v1, idx, NE), axis=-1, keepdims=True)
    n2 = jnp.where(idx == i1, -jnp.inf, noisy)
    v2m = jnp.max(n2, axis=-1, keepdims=True)
    i2 = jnp.min(jnp.where(n2 == v2m, idx, NE), axis=-1, keepdims=True)
    sel = (idx == i1) | (idx == i2)
    pr = jnp.where(sel, jnp.exp(noisy - v1), 0.0)
    pr = pr / jnp.sum(pr, axis=-1, keepdims=True)

    skip_logit = jnp.dot(xn2, ws_ref[...],
                         preferred_element_type=jnp.float32) + bs_ref[...]
    skip = jax.nn.sigmoid(skip_logit) > 0.5

    acc = jnp.zeros((P, C), jnp.float32)
    for e in range(NE):
        h = jnp.dot(xn2, we1_ref[e], preferred_element_type=jnp.float32) \
            + be1_ref[e]
        h = jnp.maximum(h, 0.0)
        ye = jnp.dot(h, we2_ref[e], preferred_element_type=jnp.float32)
        acc = acc + ye * pr[:, e:e + 1]
    acc = acc + jnp.dot(pr, be2_ref[...], preferred_element_type=jnp.float32)

    out = jnp.where(skip, x1, x1 + acc)
    out_ref[...] = out.reshape(BB, T, C)


def kernel(x, Wq, Wk, Wv, Wp, bp, ln1_g, ln1_b, ln2_g, ln2_b,
           Wer, ber, Wn, bn, Ws, bs, We1, be1, We2, be2):
    Wqf = Wq.transpose(1, 0, 2).reshape(C, NH * HD)
    Wkf = Wk.transpose(1, 0, 2).reshape(C, NH * HD)
    Wvf = Wv.transpose(1, 0, 2).reshape(C, NH * HD)
    eps = jax.random.normal(jax.random.key(42), (B, T, NE), jnp.float32)

    row = lambda a: a.reshape(1, -1)
    full = lambda arr: pl.BlockSpec(arr.shape, lambda i: (0,) * arr.ndim)

    weights = (Wqf, Wkf, Wvf, Wp, row(bp), row(ln1_g), row(ln1_b),
               row(ln2_g), row(ln2_b), Wer, row(ber), Wn, row(bn),
               Ws, row(bs), We1, be1, We2, be2)

    out = pl.pallas_call(
        _fused_kernel,
        grid=(B // BB,),
        in_specs=[pl.BlockSpec((BB, T, C), lambda i: (i, 0, 0)),
                  pl.BlockSpec((BB, T, NE), lambda i: (i, 0, 0))]
                 + [full(w) for w in weights],
        out_specs=pl.BlockSpec((BB, T, C), lambda i: (i, 0, 0)),
        out_shape=jax.ShapeDtypeStruct((B, T, C), jnp.float32),
    )(x, eps, *weights)
    return out
